# Initial kernel scaffold; baseline (speedup 1.0000x reference)
import jax
import jax.numpy as jnp
from jax.experimental import pallas as pl


def kernel(h, a_link, rollout):
    raise NotImplementedError("write your pallas kernel here")


# speedup vs baseline: 4.5324x; 4.5324x over previous
import jax
import jax.numpy as jnp
from jax.experimental import pallas as pl
from jax.experimental.pallas import tpu as pltpu

_K = 64
_N = 256
_TAU = 1.0


def _glcn_body(x_ref, xt_ref, w_ref, a_ref, p_ref):
    x = x_ref[0]
    xt = xt_ref[0]
    acc = jnp.zeros((_N, _N), jnp.float32)
    for k in range(_K):
        c = x[:, k:k + 1]
        r = xt[k:k + 1, :]
        d = jnp.abs(c - r)
        di = jax.lax.bitcast_convert_type(d, jnp.int32)
        lsb = jax.lax.shift_right_logical(di, 16) & 1
        di = (di + 0x7FFF + lsb) & jnp.int32(-65536)
        d = jax.lax.bitcast_convert_type(di, jnp.float32)
        acc = acc + w_ref[k, 0] * d
    y = jax.nn.sigmoid(acc / _TAU)
    hard = y > 0.5
    ii = jax.lax.broadcasted_iota(jnp.int32, (_N, _N), 0)
    jj = jax.lax.broadcasted_iota(jnp.int32, (_N, _N), 1)
    diag = ii == jj
    a_ref[0] = jnp.where(diag | hard, 1.0, 0.0).astype(jnp.float32)
    sel = jnp.where(hard, y, 1.0 - y)
    plog = jnp.log(sel + 1e-8)
    p_ref[...] = jnp.sum(jnp.where(diag, 0.0, plog)).reshape(1, 1, 1)


def kernel(h, a_link, rollout):
    hf = jax.lax.stop_gradient(h[:, :, :_K])
    hft = jnp.transpose(hf, (0, 2, 1))
    wi = jax.lax.bitcast_convert_type(a_link, jnp.int32)
    wlsb = jax.lax.shift_right_logical(wi, 16) & 1
    wi = (wi + 0x7FFF + wlsb) & jnp.int32(-65536)
    w_r = jax.lax.bitcast_convert_type(wi, jnp.float32)
    b = h.shape[0]
    a_out, probs = pl.pallas_call(
        _glcn_body,
        grid=(b,),
        in_specs=[
            pl.BlockSpec((1, _N, _K), lambda i: (i, 0, 0)),
            pl.BlockSpec((1, _K, _N), lambda i: (i, 0, 0)),
            pl.BlockSpec(memory_space=pltpu.SMEM),
        ],
        out_specs=[
            pl.BlockSpec((1, _N, _N), lambda i: (i, 0, 0)),
            pl.BlockSpec((1, 1, 1), lambda i: (i, 0, 0)),
        ],
        out_shape=[
            jax.ShapeDtypeStruct((b, _N, _N), jnp.float32),
            jax.ShapeDtypeStruct((b, 1, 1), jnp.float32),
        ],
    )(hf, hft, w_r)
    return (a_out, probs[:, 0, 0])


# speedup vs baseline: 6.4868x; 1.4312x over previous
import jax
import jax.numpy as jnp
from jax.experimental import pallas as pl
from jax.experimental.pallas import tpu as pltpu

_K = 64
_N = 256
_TAU = 1.0


def _glcn_body(x_ref, xt_ref, w_ref, a_ref, p_ref):
    x = x_ref[0]
    xt = xt_ref[0]
    acc = jnp.zeros((_N, _N), jnp.float32)
    for k in range(_K):
        c = x[:, k:k + 1]
        r = xt[k:k + 1, :]
        d = jnp.abs(c - r).astype(jnp.bfloat16).astype(jnp.float32)
        acc = acc + w_ref[k, 0] * d
    y = jax.nn.sigmoid(acc / _TAU)
    hard = y > 0.5
    ii = jax.lax.broadcasted_iota(jnp.int32, (_N, _N), 0)
    jj = jax.lax.broadcasted_iota(jnp.int32, (_N, _N), 1)
    diag = ii == jj
    a_ref[0] = jnp.where(diag | hard, 1.0, 0.0).astype(jnp.float32)
    sel = jnp.where(hard, y, 1.0 - y)
    plog = jnp.log(sel + 1e-8)
    p_ref[...] = jnp.sum(jnp.where(diag, 0.0, plog)).reshape(1, 1, 1)


def kernel(h, a_link, rollout):
    hf = jax.lax.stop_gradient(h[:, :, :_K])
    hft = jnp.transpose(hf, (0, 2, 1))
    wi = jax.lax.bitcast_convert_type(a_link, jnp.int32)
    wlsb = jax.lax.shift_right_logical(wi, 16) & 1
    wi = (wi + 0x7FFF + wlsb) & jnp.int32(-65536)
    w_r = jax.lax.bitcast_convert_type(wi, jnp.float32)
    b = h.shape[0]
    a_out, probs = pl.pallas_call(
        _glcn_body,
        grid=(b,),
        in_specs=[
            pl.BlockSpec((1, _N, _K), lambda i: (i, 0, 0)),
            pl.BlockSpec((1, _K, _N), lambda i: (i, 0, 0)),
            pl.BlockSpec(memory_space=pltpu.SMEM),
        ],
        out_specs=[
            pl.BlockSpec((1, _N, _N), lambda i: (i, 0, 0)),
            pl.BlockSpec((1, 1, 1), lambda i: (i, 0, 0)),
        ],
        out_shape=[
            jax.ShapeDtypeStruct((b, _N, _N), jnp.float32),
            jax.ShapeDtypeStruct((b, 1, 1), jnp.float32),
        ],
    )(hf, hft, w_r)
    return (a_out, probs[:, 0, 0])


# speedup vs baseline: 7.7578x; 1.1959x over previous
import jax
import jax.numpy as jnp
from jax.experimental import pallas as pl
from jax.experimental.pallas import tpu as pltpu

_K = 64
_N = 256
_TAU = 1.0


_H = _N // 2


def _glcn_body(x_ref, xt_ref, w_ref, a_ref, p_ref):
    x = x_ref[0]
    xt = xt_ref[0]
    a00 = jnp.zeros((_H, _H), jnp.float32)
    a01 = jnp.zeros((_H, _H), jnp.float32)
    a11 = jnp.zeros((_H, _H), jnp.float32)
    for k in range(_K):
        c0 = x[0:_H, k:k + 1]
        c1 = x[_H:_N, k:k + 1]
        r0 = xt[k:k + 1, 0:_H]
        r1 = xt[k:k + 1, _H:_N]
        wk = w_ref[k, 0]
        d00 = jnp.abs(c0 - r0).astype(jnp.bfloat16).astype(jnp.float32)
        d01 = jnp.abs(c0 - r1).astype(jnp.bfloat16).astype(jnp.float32)
        d11 = jnp.abs(c1 - r1).astype(jnp.bfloat16).astype(jnp.float32)
        a00 = a00 + wk * d00
        a01 = a01 + wk * d01
        a11 = a11 + wk * d11
    ii = jax.lax.broadcasted_iota(jnp.int32, (_H, _H), 0)
    jj = jax.lax.broadcasted_iota(jnp.int32, (_H, _H), 1)
    diag = ii == jj

    def _finish(acc, on_diag):
        y = jax.nn.sigmoid(acc / _TAU)
        hard = y > 0.5
        plog = jnp.log(jnp.where(hard, y, 1.0 - y) + 1e-8)
        if on_diag:
            a_blk = jnp.where(diag | hard, 1.0, 0.0).astype(jnp.float32)
            s = jnp.sum(jnp.where(diag, 0.0, plog))
        else:
            a_blk = jnp.where(hard, 1.0, 0.0).astype(jnp.float32)
            s = jnp.sum(plog)
        return a_blk, s

    A00, s00 = _finish(a00, True)
    A01, s01 = _finish(a01, False)
    A11, s11 = _finish(a11, True)
    a_ref[0, 0:_H, 0:_H] = A00
    a_ref[0, 0:_H, _H:_N] = A01
    a_ref[0, _H:_N, 0:_H] = jnp.transpose(A01)
    a_ref[0, _H:_N, _H:_N] = A11
    p_ref[...] = (s00 + s11 + 2.0 * s01).reshape(1, 1, 1)


def kernel(h, a_link, rollout):
    hf = jax.lax.stop_gradient(h[:, :, :_K])
    hft = jnp.transpose(hf, (0, 2, 1))
    wi = jax.lax.bitcast_convert_type(a_link, jnp.int32)
    wlsb = jax.lax.shift_right_logical(wi, 16) & 1
    wi = (wi + 0x7FFF + wlsb) & jnp.int32(-65536)
    w_r = jax.lax.bitcast_convert_type(wi, jnp.float32)
    b = h.shape[0]
    a_out, probs = pl.pallas_call(
        _glcn_body,
        grid=(b,),
        in_specs=[
            pl.BlockSpec((1, _N, _K), lambda i: (i, 0, 0)),
            pl.BlockSpec((1, _K, _N), lambda i: (i, 0, 0)),
            pl.BlockSpec(memory_space=pltpu.SMEM),
        ],
        out_specs=[
            pl.BlockSpec((1, _N, _N), lambda i: (i, 0, 0)),
            pl.BlockSpec((1, 1, 1), lambda i: (i, 0, 0)),
        ],
        out_shape=[
            jax.ShapeDtypeStruct((b, _N, _N), jnp.float32),
            jax.ShapeDtypeStruct((b, 1, 1), jnp.float32),
        ],
    )(hf, hft, w_r)
    return (a_out, probs[:, 0, 0])
